# Initial kernel scaffold; baseline (speedup 1.0000x reference)
import jax
import jax.numpy as jnp
from jax.experimental import pallas as pl


def kernel(tokens, word_table, pos_table, gamma, beta):
    raise NotImplementedError("write your pallas kernel here")


# speedup vs baseline: 2.6469x; 2.6469x over previous
import jax
import jax.numpy as jnp
from jax import lax
from jax.experimental import pallas as pl
from jax.experimental.pallas import tpu as pltpu
from jax.experimental.pallas import tpu_sc as plsc

_VOCAB = 100000
_HID = 64
_MAXLEN = 200
_BATCH = 4096
_EPS = 1e-8
_NC = 2
_NS = 16
_NW = _NC * _NS
_SEQ_PER_W = _BATCH // _NW


_GATHER_DNUMS = lax.GatherDimensionNumbers(
    offset_dims=(), collapsed_slice_dims=(0,), start_index_map=(0,))


def _permute(x, idx):
    return lax.gather(x, idx[:, None], _GATHER_DNUMS, slice_sizes=(1,),
                      mode=lax.GatherScatterMode.PROMISE_IN_BOUNDS)


def _hsum(x, perms):
    for p in perms:
        x = x + _permute(x, p)
    return x


def _rsqrt(x):
    i = lax.bitcast_convert_type(x, jnp.int32)
    y = lax.bitcast_convert_type(jnp.int32(0x5F3759DF) - (i >> 1), jnp.float32)
    for _ in range(3):
        y = y * (1.5 - 0.5 * x * y * y)
    return y


def _body(tok_hbm, word_hbm, pos_hbm, gamma_hbm, beta_hbm, out_hbm,
          pos_v, tok_v, rows_v, gamma_v, beta_v, sem):
    wid = lax.axis_index("s") * _NC + lax.axis_index("c")
    pltpu.sync_copy(pos_hbm, pos_v)
    pltpu.sync_copy(gamma_hbm, gamma_v)
    pltpu.sync_copy(beta_hbm, beta_v)
    g_regs = [gamma_v[pl.ds(16 * k, 16)] for k in range(4)]
    b_regs = [beta_v[pl.ds(16 * k, 16)] for k in range(4)]
    lane = lax.iota(jnp.int32, 16)
    perms = [lane ^ 1, lane ^ 2, lane ^ 4, lane ^ 8]

    def seq_body(s, carry):
        row = wid * _SEQ_PER_W + s
        pltpu.sync_copy(tok_hbm.at[pl.ds(row * _MAXLEN, _MAXLEN)],
                        tok_v.at[pl.ds(0, _MAXLEN)])
        c1 = pltpu.async_copy(word_hbm.at[tok_v.at[pl.ds(0, 96)]],
                              rows_v.at[pl.ds(0, 96)], sem)
        c2 = pltpu.async_copy(word_hbm.at[tok_v.at[pl.ds(96, 104)]],
                              rows_v.at[pl.ds(96, 104)], sem)
        c1.wait()
        c2.wait()

        def tok_body(t, c):
            xs = [rows_v[t, pl.ds(16 * k, 16)] + pos_v[t, pl.ds(16 * k, 16)]
                  for k in range(4)]
            ssum = (xs[0] + xs[1]) + (xs[2] + xs[3])
            mean = _hsum(ssum, perms) * (1.0 / _HID)
            d = [x - mean for x in xs]
            sq = (d[0] * d[0] + d[1] * d[1]) + (d[2] * d[2] + d[3] * d[3])
            var = _hsum(sq, perms) * (1.0 / _HID)
            rinv = _rsqrt(var + _EPS)
            tok = tok_v[pl.ds(t, 16)][0]
            maskf = jnp.where(tok != 0, jnp.float32(1.0), jnp.float32(0.0))
            a = rinv * maskf
            for k in range(4):
                rows_v[t, pl.ds(16 * k, 16)] = d[k] * (g_regs[k] * a) + b_regs[k] * maskf
            return c

        lax.fori_loop(0, _MAXLEN, tok_body, 0)
        pltpu.sync_copy(rows_v, out_hbm.at[pl.ds(row * _MAXLEN, _MAXLEN)])
        return carry

    lax.fori_loop(0, _SEQ_PER_W, seq_body, 0)


_emb = pl.kernel(
    _body,
    mesh=plsc.VectorSubcoreMesh(core_axis_name="c", subcore_axis_name="s"),
    out_type=jax.ShapeDtypeStruct((_BATCH * _MAXLEN, _HID), jnp.float32),
    scratch_types=[
        pltpu.VMEM((_MAXLEN, _HID), jnp.float32),
        pltpu.VMEM((_MAXLEN + 16,), jnp.int32),
        pltpu.VMEM((_MAXLEN, _HID), jnp.float32),
        pltpu.VMEM((_HID,), jnp.float32),
        pltpu.VMEM((_HID,), jnp.float32),
        pltpu.SemaphoreType.DMA,
    ],
    compiler_params=pltpu.CompilerParams(use_tc_tiling_on_sc=False),
)


@jax.jit
def _run(tok_flat, word_table, pos_table, gamma, beta):
    out = _emb(tok_flat, word_table, pos_table, gamma, beta)
    return out.reshape(_BATCH, _MAXLEN, _HID)


def kernel(tokens, word_table, pos_table, gamma, beta):
    tok_flat = tokens.reshape(-1).astype(jnp.int32)
    return _run(tok_flat, word_table, pos_table, gamma, beta)


# speedup vs baseline: 4.6029x; 1.7390x over previous
import jax
import jax.numpy as jnp
from jax import lax
from jax.experimental import pallas as pl
from jax.experimental.pallas import tpu as pltpu
from jax.experimental.pallas import tpu_sc as plsc

_VOCAB = 100000
_HID = 64
_MAXLEN = 200
_BATCH = 4096
_EPS = 1e-8
_NC = 2
_NS = 16
_NW = _NC * _NS
_SEQ_PER_W = _BATCH // _NW
_UNROLL = 8


_GATHER_DNUMS = lax.GatherDimensionNumbers(
    offset_dims=(), collapsed_slice_dims=(0,), start_index_map=(0,))


def _permute(x, idx):
    return lax.gather(x, idx[:, None], _GATHER_DNUMS, slice_sizes=(1,),
                      mode=lax.GatherScatterMode.PROMISE_IN_BOUNDS)


def _hsum(x, perms):
    for p in perms:
        x = x + _permute(x, p)
    return x


def _rsqrt(x):
    i = lax.bitcast_convert_type(x, jnp.int32)
    y = lax.bitcast_convert_type(jnp.int32(0x5F3759DF) - (i >> 1), jnp.float32)
    for _ in range(3):
        y = y * (1.5 - 0.5 * x * y * y)
    return y


def _body(tok_hbm, word_hbm, pos_hbm, gamma_hbm, beta_hbm, out_hbm,
          pos_v, tok_v, rows_v, gamma_v, beta_v, sem):
    wid = lax.axis_index("s") * _NC + lax.axis_index("c")
    pltpu.sync_copy(pos_hbm, pos_v)
    pltpu.sync_copy(gamma_hbm, gamma_v)
    pltpu.sync_copy(beta_hbm, beta_v)
    g_regs = [gamma_v[pl.ds(16 * k, 16)] for k in range(4)]
    b_regs = [beta_v[pl.ds(16 * k, 16)] for k in range(4)]
    lane = lax.iota(jnp.int32, 16)
    perms = [lane ^ 1, lane ^ 2, lane ^ 4, lane ^ 8]

    def seq_body(s, carry):
        row = wid * _SEQ_PER_W + s
        pltpu.sync_copy(tok_hbm.at[pl.ds(row * _MAXLEN, _MAXLEN)],
                        tok_v.at[pl.ds(0, _MAXLEN)])
        c1 = pltpu.async_copy(word_hbm.at[tok_v.at[pl.ds(0, 96)]],
                              rows_v.at[pl.ds(0, 96)], sem)
        c2 = pltpu.async_copy(word_hbm.at[tok_v.at[pl.ds(96, 104)]],
                              rows_v.at[pl.ds(96, 104)], sem)
        c1.wait()
        c2.wait()

        def tok_body(i, c):
            t0 = i * _UNROLL
            tokvec = tok_v[pl.ds(t0, 16)]
            for j in range(_UNROLL):
                t = t0 + j
                xs = [rows_v[t, pl.ds(16 * k, 16)] + pos_v[t, pl.ds(16 * k, 16)]
                      for k in range(4)]
                ssum = (xs[0] + xs[1]) + (xs[2] + xs[3])
                mean = _hsum(ssum, perms) * (1.0 / _HID)
                d = [x - mean for x in xs]
                sq = (d[0] * d[0] + d[1] * d[1]) + (d[2] * d[2] + d[3] * d[3])
                var = _hsum(sq, perms) * (1.0 / _HID)
                rinv = _rsqrt(var + _EPS)
                maskf = jnp.where(tokvec[j] != 0, jnp.float32(1.0), jnp.float32(0.0))
                a = rinv * maskf
                for k in range(4):
                    rows_v[t, pl.ds(16 * k, 16)] = (
                        d[k] * (g_regs[k] * a) + b_regs[k] * maskf)
            return c

        lax.fori_loop(0, _MAXLEN // _UNROLL, tok_body, 0)
        pltpu.sync_copy(rows_v, out_hbm.at[pl.ds(row * _MAXLEN, _MAXLEN)])
        return carry

    lax.fori_loop(0, _SEQ_PER_W, seq_body, 0)


_emb = pl.kernel(
    _body,
    mesh=plsc.VectorSubcoreMesh(core_axis_name="c", subcore_axis_name="s"),
    out_type=jax.ShapeDtypeStruct((_BATCH * _MAXLEN, _HID), jnp.float32),
    scratch_types=[
        pltpu.VMEM((_MAXLEN, _HID), jnp.float32),
        pltpu.VMEM((_MAXLEN + 16,), jnp.int32),
        pltpu.VMEM((_MAXLEN, _HID), jnp.float32),
        pltpu.VMEM((_HID,), jnp.float32),
        pltpu.VMEM((_HID,), jnp.float32),
        pltpu.SemaphoreType.DMA,
    ],
    compiler_params=pltpu.CompilerParams(use_tc_tiling_on_sc=False),
)


@jax.jit
def _run(tok_flat, word_table, pos_table, gamma, beta):
    out = _emb(tok_flat, word_table, pos_table, gamma, beta)
    return out.reshape(_BATCH, _MAXLEN, _HID)


def kernel(tokens, word_table, pos_table, gamma, beta):
    tok_flat = tokens.reshape(-1).astype(jnp.int32)
    return _run(tok_flat, word_table, pos_table, gamma, beta)
